# baseline (speedup 1.0000x reference)
import math

import jax
import jax.numpy as jnp
from jax.experimental import pallas as pl
from jax.experimental.pallas import tpu as pltpu

_V = 100000
_D = 1024
_BLK = 1024
_NBLK = (_V + _BLK - 1) // _BLK
_VPAD = _NBLK * _BLK
_EPS = 1e-05
_NEG = -1e30
_LOG_TINY = math.log(1e-30)
_T_BISECT = 17
_INT_MIN = -2147483648


def _body(h_ref, bias_ref, t_ref, tp_ref, tk_ref, g_ref, emb_ref,
          ids_ref, lp_ref, l_scr, e_scr, m_scr, n_scr):
    i = pl.program_id(0)
    h = h_ref[...]
    emb = emb_ref[...]
    lg = jax.lax.dot_general(h, emb, (((1,), (1,)), ((), ())),
                             preferred_element_type=jnp.float32)
    t = t_ref[...]
    t = jnp.where(t < _EPS, 1.0, t)
    lg = (lg + bias_ref[...]) / t
    col = i * _BLK + jax.lax.broadcasted_iota(jnp.int32, lg.shape, 1)
    lg = jnp.where(col < _V, lg, _NEG)
    l_scr[:, pl.ds(i * _BLK, _BLK)] = lg
    bmax = jnp.max(lg, axis=1, keepdims=True)
    bmin = jnp.min(jnp.where(col < _V, lg, 1e30), axis=1, keepdims=True)

    @pl.when(i == 0)
    def _init():
        m_scr[...] = bmax
        n_scr[...] = bmin

    @pl.when(i > 0)
    def _acc():
        m_scr[...] = jnp.maximum(m_scr[...], bmax)
        n_scr[...] = jnp.minimum(n_scr[...], bmin)

    @pl.when(i == _NBLK - 1)
    def _finalize():
        l = l_scr[...]
        m = jnp.maximum(m_scr[...], bmax)
        vmin = jnp.minimum(n_scr[...], bmin)
        e = jnp.exp(l - m)
        e_scr[...] = e
        s = jnp.sum(e, axis=1, keepdims=True)
        colf = jax.lax.broadcasted_iota(jnp.int32, l.shape, 1)

        tkf = tk_ref[...].astype(jnp.float32)
        tps_s = tp_ref[...] * s

        def key_of(f):
            bb = jax.lax.bitcast_convert_type(f, jnp.int32)
            return jnp.where(bb >= 0, bb, _INT_MIN - bb)

        def val_of(k):
            bb = jnp.where(k >= 0, k, _INT_MIN - k)
            return jax.lax.bitcast_convert_type(bb, jnp.float32)

        def favg(a, bb):
            return (a >> 1) + (bb >> 1) + (a & bb & 1)

        lo0 = key_of(vmin) - 1
        hi0 = key_of(m)

        def bis(_, c):
            lo, hi = c
            t2 = favg(lo, hi)
            t1 = favg(lo, t2)
            t3 = favg(t2, hi)
            ll = l_scr[...]
            ee = e_scr[...]

            def pred(tq):
                mk = ll > val_of(tq)
                cnt = jnp.sum(jnp.where(mk, 1.0, 0.0), axis=1, keepdims=True)
                ms = jnp.sum(jnp.where(mk, ee, 0.0), axis=1, keepdims=True)
                return jnp.logical_and(cnt < tkf, ms <= tps_s)

            p1 = pred(t1)
            p2 = pred(t2)
            p3 = pred(t3)
            lo2 = jnp.where(p1, lo, jnp.where(p2, t1, jnp.where(p3, t2, t3)))
            hi2 = jnp.where(p1, t1, jnp.where(p2, t2, jnp.where(p3, t3, hi)))
            return lo2, hi2

        _, hi = jax.lax.fori_loop(0, _T_BISECT, bis, (lo0, hi0))
        tau = val_of(hi)

        kept = l >= tau
        g = g_ref[...]
        z = jnp.where(kept, jnp.log(e / s + 1e-30), _LOG_TINY) + g
        zmax = jnp.max(z, axis=1, keepdims=True)
        idx = jnp.min(jnp.where(z == zmax, colf, _VPAD), axis=1, keepdims=True)
        lsel = jnp.max(jnp.where(colf == idx, l, _NEG), axis=1, keepdims=True)
        ids_ref[...] = idx
        lp_ref[...] = lsel - m - jnp.log(s)


def kernel(embedding, hidden_states, embedding_bias, temperatures, top_ps,
           top_ks):
    if hidden_states.ndim == 1:
        hidden_states = hidden_states.reshape(1, -1)
    b = hidden_states.shape[0]
    g = jax.random.gumbel(jax.random.key(1234), (b, _V), jnp.float32)
    g = jnp.pad(g, ((0, 0), (0, _VPAD - _V)))
    bias2 = jnp.pad(embedding_bias, (0, _VPAD - _V)).reshape(1, _VPAD)
    t2 = temperatures.reshape(b, 1)
    tp2 = top_ps.reshape(b, 1)
    tk2 = top_ks.reshape(b, 1)

    ids2, lp2 = pl.pallas_call(
        _body,
        grid=(_NBLK,),
        in_specs=[
            pl.BlockSpec((b, _D), lambda i: (0, 0)),
            pl.BlockSpec((1, _BLK), lambda i: (0, i)),
            pl.BlockSpec((b, 1), lambda i: (0, 0)),
            pl.BlockSpec((b, 1), lambda i: (0, 0)),
            pl.BlockSpec((b, 1), lambda i: (0, 0)),
            pl.BlockSpec((b, _VPAD), lambda i: (0, 0)),
            pl.BlockSpec((_BLK, _D), lambda i: (i, 0)),
        ],
        out_specs=[
            pl.BlockSpec((b, 1), lambda i: (0, 0)),
            pl.BlockSpec((b, 1), lambda i: (0, 0)),
        ],
        out_shape=[
            jax.ShapeDtypeStruct((b, 1), jnp.int32),
            jax.ShapeDtypeStruct((b, 1), jnp.float32),
        ],
        scratch_shapes=[
            pltpu.VMEM((b, _VPAD), jnp.float32),
            pltpu.VMEM((b, _VPAD), jnp.float32),
            pltpu.VMEM((b, 1), jnp.float32),
            pltpu.VMEM((b, 1), jnp.float32),
        ],
        compiler_params=pltpu.CompilerParams(
            dimension_semantics=("arbitrary",)),
    )(hidden_states, bias2, t2, tp2, tk2, g, embedding)
    return ids2.reshape(b), lp2.reshape(b)


# speedup vs baseline: 1.0598x; 1.0598x over previous
import math

import jax
import jax.numpy as jnp
from jax.experimental import pallas as pl
from jax.experimental.pallas import tpu as pltpu

_V = 100000
_D = 1024
_BLK = 3072
_NBLK = (_V + _BLK - 1) // _BLK
_VPAD = _NBLK * _BLK
_EPS = 1e-05
_NEG = -1e30
_LOG_TINY = math.log(1e-30)
_T_BISECT = 32
_INT_MIN = -2147483648


def _body(h_ref, bias_ref, t_ref, tp_ref, tk_ref, g_ref, emb_ref,
          ids_ref, lp_ref, l_scr, m_scr, n_scr):
    i = pl.program_id(0)
    h = h_ref[...]
    emb = emb_ref[...]
    lg = jax.lax.dot_general(h, emb, (((1,), (1,)), ((), ())),
                             preferred_element_type=jnp.float32)
    t = t_ref[...]
    t = jnp.where(t < _EPS, 1.0, t)
    lg = (lg + bias_ref[...]) / t
    col = i * _BLK + jax.lax.broadcasted_iota(jnp.int32, lg.shape, 1)
    lg = jnp.where(col < _V, lg, _NEG)
    l_scr[:, pl.ds(i * _BLK, _BLK)] = lg
    bmax = jnp.max(lg, axis=1, keepdims=True)
    bmin = jnp.min(jnp.where(col < _V, lg, 1e30), axis=1, keepdims=True)

    @pl.when(i == 0)
    def _init():
        m_scr[...] = bmax
        n_scr[...] = bmin

    @pl.when(i > 0)
    def _acc():
        m_scr[...] = jnp.maximum(m_scr[...], bmax)
        n_scr[...] = jnp.minimum(n_scr[...], bmin)

    @pl.when(i == _NBLK - 1)
    def _finalize():
        l = l_scr[...]
        m = jnp.maximum(m_scr[...], bmax)
        vmin = jnp.minimum(n_scr[...], bmin)
        s = jnp.sum(jnp.exp(l - m), axis=1, keepdims=True)
        colf = jax.lax.broadcasted_iota(jnp.int32, l.shape, 1)

        tkf = tk_ref[...].astype(jnp.float32)
        tps_s = tp_ref[...] * s

        def key_of(f):
            bb = jax.lax.bitcast_convert_type(f, jnp.int32)
            return jnp.where(bb >= 0, bb, _INT_MIN - bb)

        def val_of(k):
            bb = jnp.where(k >= 0, k, _INT_MIN - k)
            return jax.lax.bitcast_convert_type(bb, jnp.float32)

        def favg(a, bb):
            return (a >> 1) + (bb >> 1) + (a & bb & 1)

        lo0 = key_of(vmin) - 1
        hi0 = key_of(m)

        def bis(_, c):
            lo, hi = c
            mid = favg(lo, hi)
            ll = l_scr[...]
            mk = ll > val_of(mid)
            cnt = jnp.sum(jnp.where(mk, 1.0, 0.0), axis=1, keepdims=True)
            ms = jnp.sum(jnp.where(mk, jnp.exp(ll - m), 0.0), axis=1,
                         keepdims=True)
            ok = jnp.logical_and(cnt < tkf, ms <= tps_s)
            return jnp.where(ok, lo, mid), jnp.where(ok, mid, hi)

        _, hi = jax.lax.fori_loop(0, _T_BISECT, bis, (lo0, hi0))
        tau = val_of(hi)

        kept = l >= tau
        g = g_ref[...]
        z = jnp.where(kept, jnp.log(jnp.exp(l - m) / s + 1e-30),
                      _LOG_TINY) + g
        l_scr[...] = z
        zmax = jnp.max(z, axis=1, keepdims=True)
        z2 = l_scr[...]
        idx = jnp.min(jnp.where(z2 == zmax, colf, _VPAD), axis=1,
                      keepdims=True)
        gsel = jnp.max(jnp.where(colf == idx, g, _NEG), axis=1, keepdims=True)
        ids_ref[...] = idx
        lp_ref[...] = zmax - gsel


def kernel(embedding, hidden_states, embedding_bias, temperatures, top_ps,
           top_ks):
    if hidden_states.ndim == 1:
        hidden_states = hidden_states.reshape(1, -1)
    b = hidden_states.shape[0]
    g = jax.random.gumbel(jax.random.key(1234), (b, _V), jnp.float32)
    g = jnp.pad(g, ((0, 0), (0, _VPAD - _V)))
    bias2 = jnp.pad(embedding_bias, (0, _VPAD - _V)).reshape(1, _VPAD)
    t2 = temperatures.reshape(b, 1)
    tp2 = top_ps.reshape(b, 1)
    tk2 = top_ks.reshape(b, 1)

    ids2, lp2 = pl.pallas_call(
        _body,
        grid=(_NBLK,),
        in_specs=[
            pl.BlockSpec((b, _D), lambda i: (0, 0)),
            pl.BlockSpec((1, _BLK), lambda i: (0, i)),
            pl.BlockSpec((b, 1), lambda i: (0, 0)),
            pl.BlockSpec((b, 1), lambda i: (0, 0)),
            pl.BlockSpec((b, 1), lambda i: (0, 0)),
            pl.BlockSpec((b, _VPAD), lambda i: (0, 0)),
            pl.BlockSpec((_BLK, _D), lambda i: (i, 0)),
        ],
        out_specs=[
            pl.BlockSpec((b, 1), lambda i: (0, 0)),
            pl.BlockSpec((b, 1), lambda i: (0, 0)),
        ],
        out_shape=[
            jax.ShapeDtypeStruct((b, 1), jnp.int32),
            jax.ShapeDtypeStruct((b, 1), jnp.float32),
        ],
        scratch_shapes=[
            pltpu.VMEM((b, _VPAD), jnp.float32),
            pltpu.VMEM((b, 1), jnp.float32),
            pltpu.VMEM((b, 1), jnp.float32),
        ],
        compiler_params=pltpu.CompilerParams(
            dimension_semantics=("arbitrary",)),
    )(hidden_states, bias2, t2, tp2, tk2, g, embedding)
    return ids2.reshape(b), lp2.reshape(b)


# speedup vs baseline: 1.1948x; 1.1274x over previous
import math

import jax
import jax.numpy as jnp
from jax.experimental import pallas as pl
from jax.experimental.pallas import tpu as pltpu

_V = 100000
_D = 1024
_BLK = 3072
_NBLK = (_V + _BLK - 1) // _BLK
_VPAD = _NBLK * _BLK
_EPS = 1e-05
_NEG = -1e30
_LOG_TINY = math.log(1e-30)
_T_BISECT = 32
_INT_MIN = -2147483648


def _body(h_ref, bias_ref, t_ref, tp_ref, tk_ref, g_ref, emb_ref,
          ids_ref, lp_ref, l_scr, e_scr, m_scr, n_scr):
    i = pl.program_id(0)
    h = h_ref[...]
    emb = emb_ref[...]
    lg = jax.lax.dot_general(h, emb, (((1,), (1,)), ((), ())),
                             preferred_element_type=jnp.float32)
    t = t_ref[...]
    t = jnp.where(t < _EPS, 1.0, t)
    lg = (lg + bias_ref[...]) / t
    col = i * _BLK + jax.lax.broadcasted_iota(jnp.int32, lg.shape, 1)
    lg = jnp.where(col < _V, lg, _NEG)
    l_scr[:, pl.ds(i * _BLK, _BLK)] = lg
    bmax = jnp.max(lg, axis=1, keepdims=True)
    bmin = jnp.min(jnp.where(col < _V, lg, 1e30), axis=1, keepdims=True)

    @pl.when(i == 0)
    def _init():
        m_scr[...] = bmax
        n_scr[...] = bmin

    @pl.when(i > 0)
    def _acc():
        m_scr[...] = jnp.maximum(m_scr[...], bmax)
        n_scr[...] = jnp.minimum(n_scr[...], bmin)

    @pl.when(i == _NBLK - 1)
    def _finalize():
        l = l_scr[...]
        m = jnp.maximum(m_scr[...], bmax)
        vmin = jnp.minimum(n_scr[...], bmin)
        e = jnp.exp(l - m)
        e_scr[...] = e
        s = jnp.sum(e, axis=1, keepdims=True)
        colf = jax.lax.broadcasted_iota(jnp.int32, l.shape, 1)

        tkf = tk_ref[...].astype(jnp.float32)
        tps_s = tp_ref[...] * s

        def key_of(f):
            bb = jax.lax.bitcast_convert_type(f, jnp.int32)
            return jnp.where(bb >= 0, bb, _INT_MIN - bb)

        def val_of(k):
            bb = jnp.where(k >= 0, k, _INT_MIN - k)
            return jax.lax.bitcast_convert_type(bb, jnp.float32)

        def favg(a, bb):
            return (a >> 1) + (bb >> 1) + (a & bb & 1)

        lo0 = key_of(vmin) - 1
        hi0 = key_of(m)

        def bis(_, c):
            lo, hi = c
            mid = favg(lo, hi)
            ll = l_scr[...]
            mk = ll > val_of(mid)
            cnt = jnp.sum(jnp.where(mk, 1.0, 0.0), axis=1, keepdims=True)
            ms = jnp.sum(jnp.where(mk, e_scr[...], 0.0), axis=1,
                         keepdims=True)
            ok = jnp.logical_and(cnt < tkf, ms <= tps_s)
            return jnp.where(ok, lo, mid), jnp.where(ok, mid, hi)

        _, hi = jax.lax.fori_loop(0, _T_BISECT, bis, (lo0, hi0))
        tau = val_of(hi)

        kept = l >= tau
        g = g_ref[...]
        z = jnp.where(kept, jnp.log(e_scr[...] / s + 1e-30),
                      _LOG_TINY) + g
        l_scr[...] = z
        zmax = jnp.max(z, axis=1, keepdims=True)
        z2 = l_scr[...]
        idx = jnp.min(jnp.where(z2 == zmax, colf, _VPAD), axis=1,
                      keepdims=True)
        gsel = jnp.max(jnp.where(colf == idx, g, _NEG), axis=1, keepdims=True)
        ids_ref[...] = idx
        lp_ref[...] = zmax - gsel


def kernel(embedding, hidden_states, embedding_bias, temperatures, top_ps,
           top_ks):
    if hidden_states.ndim == 1:
        hidden_states = hidden_states.reshape(1, -1)
    b = hidden_states.shape[0]
    g = jax.random.gumbel(jax.random.key(1234), (b, _V), jnp.float32)
    g = jnp.pad(g, ((0, 0), (0, _VPAD - _V)))
    bias2 = jnp.pad(embedding_bias, (0, _VPAD - _V)).reshape(1, _VPAD)
    t2 = temperatures.reshape(b, 1)
    tp2 = top_ps.reshape(b, 1)
    tk2 = top_ks.reshape(b, 1)

    ids2, lp2 = pl.pallas_call(
        _body,
        grid=(_NBLK,),
        in_specs=[
            pl.BlockSpec((b, _D), lambda i: (0, 0)),
            pl.BlockSpec((1, _BLK), lambda i: (0, i)),
            pl.BlockSpec((b, 1), lambda i: (0, 0)),
            pl.BlockSpec((b, 1), lambda i: (0, 0)),
            pl.BlockSpec((b, 1), lambda i: (0, 0)),
            pl.BlockSpec((b, _VPAD), lambda i: (0, 0)),
            pl.BlockSpec((_BLK, _D), lambda i: (i, 0)),
        ],
        out_specs=[
            pl.BlockSpec((b, 1), lambda i: (0, 0)),
            pl.BlockSpec((b, 1), lambda i: (0, 0)),
        ],
        out_shape=[
            jax.ShapeDtypeStruct((b, 1), jnp.int32),
            jax.ShapeDtypeStruct((b, 1), jnp.float32),
        ],
        scratch_shapes=[
            pltpu.VMEM((b, _VPAD), jnp.float32),
            pltpu.VMEM((b, _VPAD), jnp.float32),
            pltpu.VMEM((b, 1), jnp.float32),
            pltpu.VMEM((b, 1), jnp.float32),
        ],
        compiler_params=pltpu.CompilerParams(
            dimension_semantics=("arbitrary",)),
    )(hidden_states, bias2, t2, tp2, tk2, g, embedding)
    return ids2.reshape(b), lp2.reshape(b)


# speedup vs baseline: 1.3644x; 1.1419x over previous
import math

import jax
import jax.numpy as jnp
from jax.experimental import pallas as pl
from jax.experimental.pallas import tpu as pltpu

_V = 100000
_D = 1024
_BLK = 3072
_NBLK = (_V + _BLK - 1) // _BLK
_VPAD = _NBLK * _BLK
_EPS = 1e-05
_NEG = -1e30
_LOG_TINY = math.log(1e-30)
_T_BISECT = 32
_INT_MIN = -2147483648
_LOG_V = math.log(_V)

_KS0 = 0
_KS1 = 1234
_KS2 = (_KS0 ^ _KS1 ^ 0x1BD11BDA)
_ROTS = (13, 15, 26, 6, 17, 29, 16, 24)
_TINY = float(jnp.finfo(jnp.float32).tiny)


def _i32(x):
    return jnp.int32(x)


def _gumbel_block(jj):
    srl = jax.lax.shift_right_logical
    ks = (_i32(_KS0), _i32(_KS1), _i32(_KS2))
    x0 = jnp.zeros_like(jj) + ks[0]
    x1 = jj + ks[1]
    for d in range(5):
        for q in range(4):
            r = _ROTS[(d % 2) * 4 + q]
            x0 = x0 + x1
            x1 = jax.lax.shift_left(x1, _i32(r)) | srl(x1, _i32(32 - r))
            x1 = x1 ^ x0
        x0 = x0 + ks[(d + 1) % 3]
        x1 = x1 + ks[(d + 2) % 3] + _i32(d + 1)
    bits = x0 ^ x1
    fb = srl(bits, _i32(9)) | _i32(0x3F800000)
    fl = jax.lax.bitcast_convert_type(fb, jnp.float32) - 1.0
    u = jnp.maximum(_TINY, fl + _TINY)
    return -jnp.log(-jnp.log(u))


def _body(h_ref, bias_ref, t_ref, tp_ref, tk_ref, emb_ref,
          ids_ref, lp_ref, l_scr, e_scr, g_scr, m_scr, n_scr):
    i = pl.program_id(0)
    h = h_ref[...]
    emb = emb_ref[...]
    lg = jax.lax.dot_general(h, emb, (((1,), (1,)), ((), ())),
                             preferred_element_type=jnp.float32)
    t = t_ref[...]
    t = jnp.where(t < _EPS, 1.0, t)
    lg = (lg + bias_ref[...]) / t
    col = i * _BLK + jax.lax.broadcasted_iota(jnp.int32, lg.shape, 1)
    lg = jnp.where(col < _V, lg, _NEG)
    l_scr[:, pl.ds(i * _BLK, _BLK)] = lg
    row = jax.lax.broadcasted_iota(jnp.int32, lg.shape, 0)
    gg = _gumbel_block(row * _V + col)
    g_scr[:, pl.ds(i * _BLK, _BLK)] = jnp.where(col < _V, gg, 0.0)
    bmax = jnp.max(lg, axis=1, keepdims=True)
    bmin = jnp.min(jnp.where(col < _V, lg, 1e30), axis=1, keepdims=True)

    @pl.when(i == 0)
    def _init():
        m_scr[...] = bmax
        n_scr[...] = bmin

    @pl.when(i > 0)
    def _acc():
        m_scr[...] = jnp.maximum(m_scr[...], bmax)
        n_scr[...] = jnp.minimum(n_scr[...], bmin)

    @pl.when(i == _NBLK - 1)
    def _finalize():
        l = l_scr[...]
        m = jnp.maximum(m_scr[...], bmax)
        vmin = jnp.minimum(n_scr[...], bmin)
        e = jnp.exp(l - m)
        e_scr[...] = e
        s = jnp.sum(e, axis=1, keepdims=True)
        colf = jax.lax.broadcasted_iota(jnp.int32, l.shape, 1)

        tkf = tk_ref[...].astype(jnp.float32)
        tps_s = tp_ref[...] * s

        def key_of(f):
            bb = jax.lax.bitcast_convert_type(f, jnp.int32)
            return jnp.where(bb >= 0, bb, _INT_MIN - bb)

        def val_of(k):
            bb = jnp.where(k >= 0, k, _INT_MIN - k)
            return jax.lax.bitcast_convert_type(bb, jnp.float32)

        def favg(a, bb):
            return (a >> 1) + (bb >> 1) + (a & bb & 1)

        lo_val = m + jnp.log(s * (1.0 - tp_ref[...])) - _LOG_V - 0.1
        lo0 = jnp.maximum(key_of(vmin) - 1, key_of(lo_val))
        hi0 = key_of(m)

        def bis(_, c):
            lo, hi = c
            mid = favg(lo, hi)
            ll = l_scr[...]
            mk = ll > val_of(mid)
            cnt = jnp.sum(jnp.where(mk, 1.0, 0.0), axis=1, keepdims=True)
            ms = jnp.sum(jnp.where(mk, e_scr[...], 0.0), axis=1,
                         keepdims=True)
            ok = jnp.logical_and(cnt < tkf, ms <= tps_s)
            return jnp.where(ok, lo, mid), jnp.where(ok, mid, hi)

        _, hi = jax.lax.fori_loop(0, _T_BISECT, bis, (lo0, hi0))
        tau = val_of(hi)

        kept = l >= tau
        g = g_scr[...]
        z = jnp.where(kept, jnp.log(e_scr[...] / s + 1e-30),
                      _LOG_TINY) + g
        l_scr[...] = z
        zmax = jnp.max(z, axis=1, keepdims=True)
        z2 = l_scr[...]
        idx = jnp.min(jnp.where(z2 == zmax, colf, _VPAD), axis=1,
                      keepdims=True)
        gsel = jnp.max(jnp.where(colf == idx, g, _NEG), axis=1, keepdims=True)
        ids_ref[...] = idx
        lp_ref[...] = zmax - gsel


def kernel(embedding, hidden_states, embedding_bias, temperatures, top_ps,
           top_ks):
    if hidden_states.ndim == 1:
        hidden_states = hidden_states.reshape(1, -1)
    b = hidden_states.shape[0]
    bias2 = jnp.pad(embedding_bias, (0, _VPAD - _V)).reshape(1, _VPAD)
    t2 = temperatures.reshape(b, 1)
    tp2 = top_ps.reshape(b, 1)
    tk2 = top_ks.reshape(b, 1)

    ids2, lp2 = pl.pallas_call(
        _body,
        grid=(_NBLK,),
        in_specs=[
            pl.BlockSpec((b, _D), lambda i: (0, 0)),
            pl.BlockSpec((1, _BLK), lambda i: (0, i)),
            pl.BlockSpec((b, 1), lambda i: (0, 0)),
            pl.BlockSpec((b, 1), lambda i: (0, 0)),
            pl.BlockSpec((b, 1), lambda i: (0, 0)),
            pl.BlockSpec((_BLK, _D), lambda i: (i, 0)),
        ],
        out_specs=[
            pl.BlockSpec((b, 1), lambda i: (0, 0)),
            pl.BlockSpec((b, 1), lambda i: (0, 0)),
        ],
        out_shape=[
            jax.ShapeDtypeStruct((b, 1), jnp.int32),
            jax.ShapeDtypeStruct((b, 1), jnp.float32),
        ],
        scratch_shapes=[
            pltpu.VMEM((b, _VPAD), jnp.float32),
            pltpu.VMEM((b, _VPAD), jnp.float32),
            pltpu.VMEM((b, _VPAD), jnp.float32),
            pltpu.VMEM((b, 1), jnp.float32),
            pltpu.VMEM((b, 1), jnp.float32),
        ],
        compiler_params=pltpu.CompilerParams(
            dimension_semantics=("arbitrary",)),
    )(hidden_states, bias2, t2, tp2, tk2, embedding)
    return ids2.reshape(b), lp2.reshape(b)


# speedup vs baseline: 1.3978x; 1.0245x over previous
import math

import jax
import jax.numpy as jnp
from jax.experimental import pallas as pl
from jax.experimental.pallas import tpu as pltpu

_V = 100000
_D = 1024
_BLK = 3072
_NBLK = (_V + _BLK - 1) // _BLK
_VPAD = _NBLK * _BLK
_EPS = 1e-05
_NEG = -1e30
_LOG_TINY = math.log(1e-30)
_T_BISECT = 30

_KS0 = 0
_KS1 = 1234
_KS2 = (_KS0 ^ _KS1 ^ 0x1BD11BDA)
_ROTS = (13, 15, 26, 6, 17, 29, 16, 24)
_TINY = float(jnp.finfo(jnp.float32).tiny)


def _i32(x):
    return jnp.int32(x)


def _gumbel_block(jj):
    srl = jax.lax.shift_right_logical
    ks = (_i32(_KS0), _i32(_KS1), _i32(_KS2))
    x0 = jnp.zeros_like(jj) + ks[0]
    x1 = jj + ks[1]
    for d in range(5):
        for q in range(4):
            r = _ROTS[(d % 2) * 4 + q]
            x0 = x0 + x1
            x1 = jax.lax.shift_left(x1, _i32(r)) | srl(x1, _i32(32 - r))
            x1 = x1 ^ x0
        x0 = x0 + ks[(d + 1) % 3]
        x1 = x1 + ks[(d + 2) % 3] + _i32(d + 1)
    bits = x0 ^ x1
    fb = srl(bits, _i32(9)) | _i32(0x3F800000)
    fl = jax.lax.bitcast_convert_type(fb, jnp.float32) - 1.0
    u = jnp.maximum(_TINY, fl + _TINY)
    return -jnp.log(-jnp.log(u))


def _body(h_ref, bias_ref, t_ref, tp_ref, tk_ref, emb_ref,
          ids_ref, lp_ref, l_scr, e_scr, g_scr, m_scr):
    i = pl.program_id(0)
    h = h_ref[...]
    emb = emb_ref[...]
    lg = jax.lax.dot_general(h, emb, (((1,), (1,)), ((), ())),
                             preferred_element_type=jnp.float32)
    t = t_ref[...]
    t = jnp.where(t < _EPS, 1.0, t)
    lg = (lg + bias_ref[...]) / t
    col = i * _BLK + jax.lax.broadcasted_iota(jnp.int32, lg.shape, 1)
    lg = jnp.where(col < _V, lg, _NEG)
    l_scr[:, pl.ds(i * _BLK, _BLK)] = lg
    row = jax.lax.broadcasted_iota(jnp.int32, lg.shape, 0)
    gg = _gumbel_block(row * _V + col)
    g_scr[:, pl.ds(i * _BLK, _BLK)] = jnp.where(col < _V, gg, 0.0)
    bmax = jnp.max(lg, axis=1, keepdims=True)

    @pl.when(i == 0)
    def _init():
        m_scr[...] = bmax

    @pl.when(i > 0)
    def _acc():
        m_scr[...] = jnp.maximum(m_scr[...], bmax)

    @pl.when(i == _NBLK - 1)
    def _finalize():
        l = l_scr[...]
        m = jnp.maximum(m_scr[...], bmax)
        e = jnp.exp(l - m)
        e_scr[...] = e
        s = jnp.sum(e, axis=1, keepdims=True)
        colf = jax.lax.broadcasted_iota(jnp.int32, l.shape, 1)

        tkf = tk_ref[...].astype(jnp.float32)
        tps_s = tp_ref[...] * s

        lo0 = jnp.full(m.shape, -1, jnp.int32)
        hi0 = jnp.full(m.shape, 0x3F800000, jnp.int32)

        def bis(_, c):
            lo, hi = c
            tq = (lo + hi) >> 1
            ee = e_scr[...]
            ebits = jax.lax.bitcast_convert_type(ee, jnp.int32)
            mk = ebits > tq
            cnt = jnp.sum(jnp.where(mk, 1.0, 0.0), axis=1, keepdims=True)
            ms = jnp.sum(jnp.where(mk, ee, 0.0), axis=1, keepdims=True)
            ok = jnp.logical_and(cnt < tkf, ms <= tps_s)
            return jnp.where(ok, lo, tq), jnp.where(ok, tq, hi)

        _, hi = jax.lax.fori_loop(0, _T_BISECT, bis, (lo0, hi0))

        ee = e_scr[...]
        kept = jax.lax.bitcast_convert_type(ee, jnp.int32) >= hi
        g = g_scr[...]
        z = jnp.where(kept, jnp.log(ee / s + 1e-30), _LOG_TINY) + g
        l_scr[...] = z
        zmax = jnp.max(z, axis=1, keepdims=True)
        z2 = l_scr[...]
        idx = jnp.min(jnp.where(z2 == zmax, colf, _VPAD), axis=1,
                      keepdims=True)
        gsel = jnp.max(jnp.where(colf == idx, g, _NEG), axis=1, keepdims=True)
        ids_ref[...] = idx
        lp_ref[...] = zmax - gsel


def kernel(embedding, hidden_states, embedding_bias, temperatures, top_ps,
           top_ks):
    if hidden_states.ndim == 1:
        hidden_states = hidden_states.reshape(1, -1)
    b = hidden_states.shape[0]
    bias2 = jnp.pad(embedding_bias, (0, _VPAD - _V)).reshape(1, _VPAD)
    t2 = temperatures.reshape(b, 1)
    tp2 = top_ps.reshape(b, 1)
    tk2 = top_ks.reshape(b, 1)

    ids2, lp2 = pl.pallas_call(
        _body,
        grid=(_NBLK,),
        in_specs=[
            pl.BlockSpec((b, _D), lambda i: (0, 0)),
            pl.BlockSpec((1, _BLK), lambda i: (0, i)),
            pl.BlockSpec((b, 1), lambda i: (0, 0)),
            pl.BlockSpec((b, 1), lambda i: (0, 0)),
            pl.BlockSpec((b, 1), lambda i: (0, 0)),
            pl.BlockSpec((_BLK, _D), lambda i: (i, 0)),
        ],
        out_specs=[
            pl.BlockSpec((b, 1), lambda i: (0, 0)),
            pl.BlockSpec((b, 1), lambda i: (0, 0)),
        ],
        out_shape=[
            jax.ShapeDtypeStruct((b, 1), jnp.int32),
            jax.ShapeDtypeStruct((b, 1), jnp.float32),
        ],
        scratch_shapes=[
            pltpu.VMEM((b, _VPAD), jnp.float32),
            pltpu.VMEM((b, _VPAD), jnp.float32),
            pltpu.VMEM((b, _VPAD), jnp.float32),
            pltpu.VMEM((b, 1), jnp.float32),
        ],
        compiler_params=pltpu.CompilerParams(
            dimension_semantics=("arbitrary",)),
    )(hidden_states, bias2, t2, tp2, tk2, embedding)
    return ids2.reshape(b), lp2.reshape(b)
